# Initial kernel scaffold; baseline (speedup 1.0000x reference)
import jax
import jax.numpy as jnp
from jax.experimental import pallas as pl


def kernel(x, edge_index, W1, b1, W2, b2):
    raise NotImplementedError("write your pallas kernel here")


# speedup vs baseline: 14.3124x; 14.3124x over previous
import functools

import jax
import jax.numpy as jnp
from jax import lax
from jax.experimental import pallas as pl
from jax.experimental.pallas import tpu as pltpu
from jax.experimental.pallas import tpu_sc as plsc

N = 10000
D = 128
E = 320000

NC = 2
NS = 16
NW = NC * NS
CHUNK = 128
NCH = -(-E // (NW * CHUNK))
EPAD = NW * NCH * CHUNK
NPAD = 10240
ROWS_PER_TILE = NPAD // NS

_mesh = plsc.VectorSubcoreMesh(
    core_axis_name="c", subcore_axis_name="s", num_cores=NC, num_subcores=NS
)


@functools.partial(
    pl.kernel,
    out_type=jax.ShapeDtypeStruct((NC, NPAD), jnp.float32),
    mesh=_mesh,
    scratch_types=[
        pltpu.VMEM_SHARED((NPAD,), jnp.float32),
        pltpu.VMEM((NCH, CHUNK), jnp.int32),
        pltpu.VMEM((CHUNK,), jnp.float32),
        pltpu.VMEM((ROWS_PER_TILE,), jnp.float32),
    ],
)
def _sc_deg(dst_hbm, out_hbm, acc_sp, dst_v, ones_v, zb_v):
    c = lax.axis_index("c")
    s = lax.axis_index("s")
    wid = s * NC + c

    def zfill(i, _):
        zb_v[pl.ds(i * 16, 16)] = jnp.zeros((16,), jnp.float32)
        return 0

    lax.fori_loop(0, ROWS_PER_TILE // 16, zfill, 0)

    def ofill(i, _):
        ones_v[pl.ds(i * 16, 16)] = jnp.ones((16,), jnp.float32)
        return 0

    lax.fori_loop(0, CHUNK // 16, ofill, 0)

    pltpu.sync_copy(zb_v, acc_sp.at[pl.ds(s * ROWS_PER_TILE, ROWS_PER_TILE)])
    plsc.subcore_barrier()

    pltpu.sync_copy(dst_hbm.at[wid], dst_v)

    def body(j, _):
        pltpu.sync_copy(ones_v, acc_sp.at[dst_v.at[j]], add=True)
        return 0

    lax.fori_loop(0, NCH, body, 0)
    plsc.subcore_barrier()
    pltpu.sync_copy(
        acc_sp.at[pl.ds(s * ROWS_PER_TILE, ROWS_PER_TILE)],
        out_hbm.at[c, pl.ds(s * ROWS_PER_TILE, ROWS_PER_TILE)],
    )


@functools.partial(
    pl.kernel,
    out_type=jax.ShapeDtypeStruct((NC, NPAD, D), jnp.float32),
    mesh=_mesh,
    scratch_types=[
        pltpu.VMEM_SHARED((NPAD, D), jnp.float32),
        pltpu.VMEM((NCH, CHUNK), jnp.int32),
        pltpu.VMEM((NCH, CHUNK), jnp.int32),
        pltpu.VMEM((CHUNK, D), jnp.float32),
        pltpu.SemaphoreType.DMA,
    ],
)
def _sc_agg(table_hbm, src_hbm, dst_hbm, out_hbm, acc_sp, src_v, dst_v, rows_v, sem):
    c = lax.axis_index("c")
    s = lax.axis_index("s")
    wid = s * NC + c

    def zr(i, _):
        for k in range(D // 16):
            rows_v[i, pl.ds(k * 16, 16)] = jnp.zeros((16,), jnp.float32)
        return 0

    lax.fori_loop(0, CHUNK, zr, 0)

    def zs(k, _):
        pltpu.sync_copy(
            rows_v, acc_sp.at[pl.ds(s * ROWS_PER_TILE + k * CHUNK, CHUNK)]
        )
        return 0

    lax.fori_loop(0, ROWS_PER_TILE // CHUNK, zs, 0)
    plsc.subcore_barrier()

    pltpu.sync_copy(src_hbm.at[wid], src_v)
    pltpu.sync_copy(dst_hbm.at[wid], dst_v)

    def body(j, _):
        pltpu.async_copy(table_hbm.at[src_v.at[j]], rows_v, sem).wait()
        pltpu.sync_copy(rows_v, acc_sp.at[dst_v.at[j]], add=True)
        return 0

    lax.fori_loop(0, NCH, body, 0)
    plsc.subcore_barrier()

    def co(k, _):
        pltpu.sync_copy(
            acc_sp.at[pl.ds(s * ROWS_PER_TILE + k * CHUNK, CHUNK)],
            out_hbm.at[c, pl.ds(s * ROWS_PER_TILE + k * CHUNK, CHUNK)],
        )
        return 0

    lax.fori_loop(0, ROWS_PER_TILE // CHUNK, co, 0)


_BR = 1024


def _dis_of(deg_ref):
    deg = deg_ref[0, :] + deg_ref[1, :] + 1.0
    return lax.rsqrt(deg)


def _tc1_body(deg_ref, x_ref, w_ref, o_ref):
    dis = _dis_of(deg_ref)
    xw = jnp.dot(x_ref[...], w_ref[...], preferred_element_type=jnp.float32)
    o_ref[...] = xw * dis[:, None]


def _tc2_body(deg_ref, acc_ref, xws_ref, b_ref, w_ref, h_ref, o_ref):
    dis = _dis_of(deg_ref)
    h = jnp.maximum(
        dis[:, None] * (acc_ref[0] + acc_ref[1] + xws_ref[...]) + b_ref[...], 0.0
    )
    h_ref[...] = h
    o_ref[...] = jnp.dot(h, w_ref[...], preferred_element_type=jnp.float32) * dis[:, None]


def _tc3_body(deg_ref, acc_ref, xws_ref, b_ref, h1_ref, o_ref):
    dis = _dis_of(deg_ref)
    h2 = jnp.maximum(
        dis[:, None] * (acc_ref[0] + acc_ref[1] + xws_ref[...]) + b_ref[...], 0.0
    )
    o_ref[:, :D] = h1_ref[...]
    o_ref[:, D:] = h2


_deg_spec = pl.BlockSpec((NC, _BR), lambda i: (0, i))
_row_spec = pl.BlockSpec((_BR, D), lambda i: (i, 0))
_acc_spec = pl.BlockSpec((NC, _BR, D), lambda i: (0, i, 0))
_w_spec = pl.BlockSpec((D, D), lambda i: (0, 0))
_b_spec = pl.BlockSpec((1, D), lambda i: (0, 0))
_grid = (NPAD // _BR,)

_tc1 = pl.pallas_call(
    _tc1_body,
    grid=_grid,
    in_specs=[_deg_spec, _row_spec, _w_spec],
    out_specs=_row_spec,
    out_shape=jax.ShapeDtypeStruct((NPAD, D), jnp.float32),
)

_tc2 = pl.pallas_call(
    _tc2_body,
    grid=_grid,
    in_specs=[_deg_spec, _acc_spec, _row_spec, _b_spec, _w_spec],
    out_specs=[_row_spec, _row_spec],
    out_shape=[
        jax.ShapeDtypeStruct((NPAD, D), jnp.float32),
        jax.ShapeDtypeStruct((NPAD, D), jnp.float32),
    ],
)

_tc3 = pl.pallas_call(
    _tc3_body,
    grid=_grid,
    in_specs=[_deg_spec, _acc_spec, _row_spec, _b_spec, _row_spec],
    out_specs=pl.BlockSpec((_BR, 2 * D), lambda i: (i, 0)),
    out_shape=jax.ShapeDtypeStruct((NPAD, 2 * D), jnp.float32),
)


@jax.jit
def kernel(x, edge_index, W1, b1, W2, b2):
    src = edge_index[0].astype(jnp.int32)
    dst = edge_index[1].astype(jnp.int32)
    pad = EPAD - E
    padv = jnp.full((pad,), N, jnp.int32)
    src_r = jnp.concatenate([src, padv]).reshape(NW, NCH, CHUNK)
    dst_r = jnp.concatenate([dst, padv]).reshape(NW, NCH, CHUNK)
    x_pad = jnp.pad(x, ((0, NPAD - N), (0, 0)))
    b1r = b1.reshape(1, D)
    b2r = b2.reshape(1, D)

    deg2 = _sc_deg(dst_r)
    xw1s = _tc1(deg2, x_pad, W1)
    acc1 = _sc_agg(xw1s, src_r, dst_r)
    h1p, xw2s = _tc2(deg2, acc1, xw1s, b1r, W2)
    acc2 = _sc_agg(xw2s, src_r, dst_r)
    out = _tc3(deg2, acc2, xw2s, b2r, h1p)
    return out[:N]
